# baseline (device time: 18139 ns/iter reference)
import jax
import jax.numpy as jnp
from jax import lax
from jax.experimental import pallas as pl
from jax.experimental.pallas import tpu as pltpu

N_DEV = 32


def kernel(x, dy, gamma):
    m, d = x.shape

    def body(x_ref, dy_ref, out_ref, acc_ref, comm_ref, send_sems, recv_sems):
        my = lax.axis_index("i")

        barrier_sem = pltpu.get_barrier_semaphore()
        for off in range(1, N_DEV):
            p = lax.rem(my + off, N_DEV)
            pl.semaphore_signal(barrier_sem, inc=1, device_id=(p,),
                                device_id_type=pl.DeviceIdType.MESH)

        xv = x_ref[:, :]
        dyv = dy_ref[:, :]
        mu = jnp.mean(xv, axis=1, keepdims=True)
        xc = xv - mu
        var = jnp.mean(xc * xc, axis=1, keepdims=True)
        rstd = lax.rsqrt(var + 1e-5)
        dg = jnp.sum(dyv * (xc * rstd), axis=0, keepdims=True)
        db = jnp.sum(dyv, axis=0, keepdims=True)
        acc_ref[:, :] = jnp.concatenate([dg, db], axis=0)

        pl.semaphore_wait(barrier_sem, N_DEV - 1)

        rdmas = []
        for off in range(1, N_DEV):
            p = lax.rem(my + off, N_DEV)
            rdma = pltpu.make_async_remote_copy(
                src_ref=acc_ref,
                dst_ref=comm_ref.at[my],
                send_sem=send_sems.at[p],
                recv_sem=recv_sems.at[my],
                device_id=(p,),
                device_id_type=pl.DeviceIdType.MESH,
            )
            rdma.start()
            rdmas.append(rdma)

        comm_ref[my, :, :] = acc_ref[:, :]

        for off in range(1, N_DEV):
            q = lax.rem(my + off, N_DEV)
            recv = pltpu.make_async_remote_copy(
                src_ref=acc_ref,
                dst_ref=comm_ref.at[q],
                send_sem=send_sems.at[q],
                recv_sem=recv_sems.at[q],
                device_id=(q,),
                device_id_type=pl.DeviceIdType.MESH,
            )
            recv.wait_recv()

        out_ref[:, :] = jnp.sum(comm_ref[:, :, :], axis=0)

        for rdma in rdmas:
            rdma.wait_send()

    return pl.pallas_call(
        body,
        out_shape=jax.ShapeDtypeStruct((2, d), jnp.float32),
        in_specs=[
            pl.BlockSpec(memory_space=pltpu.VMEM),
            pl.BlockSpec(memory_space=pltpu.VMEM),
        ],
        out_specs=pl.BlockSpec(memory_space=pltpu.VMEM),
        scratch_shapes=[
            pltpu.VMEM((2, d), jnp.float32),
            pltpu.VMEM((N_DEV, 2, d), jnp.float32),
            pltpu.SemaphoreType.DMA((N_DEV,)),
            pltpu.SemaphoreType.DMA((N_DEV,)),
        ],
        compiler_params=pltpu.CompilerParams(collective_id=0),
    )(x, dy)


# device time: 6784 ns/iter; 2.6738x vs baseline; 2.6738x over previous
import jax
import jax.numpy as jnp
from jax import lax
from jax.experimental import pallas as pl
from jax.experimental.pallas import tpu as pltpu

N_DEV = 32
G = 8


def kernel(x, dy, gamma):
    m, d = x.shape
    mb = m // G

    def body(x_ref, dy_ref, out_ref, acc_ref, comm_ref, send_sems, recv_sems):
        my = lax.axis_index("i")
        g = pl.program_id(0)
        barrier_sem = pltpu.get_barrier_semaphore()

        @pl.when(g == 0)
        def _signal():
            for off in range(1, N_DEV):
                p = lax.rem(my + off, N_DEV)
                pl.semaphore_signal(barrier_sem, inc=1, device_id=(p,),
                                    device_id_type=pl.DeviceIdType.MESH)

        xv = x_ref[:, :]
        dyv = dy_ref[:, :]
        s1 = jnp.sum(xv, axis=1, keepdims=True)
        s2 = jnp.sum(xv * xv, axis=1, keepdims=True)
        mu = s1 * (1.0 / d)
        var = s2 * (1.0 / d) - mu * mu
        rstd = lax.rsqrt(var + 1e-5)
        t = dyv * (rstd * xv - mu * rstd)
        dg = jnp.sum(t, axis=0, keepdims=True)
        db = jnp.sum(dyv, axis=0, keepdims=True)
        blk = jnp.concatenate([dg, db], axis=0)
        prev = jnp.where(g == 0, jnp.zeros_like(blk), acc_ref[:, :])
        acc_ref[:, :] = prev + blk

        @pl.when(g == G - 1)
        def _exchange():
            pl.semaphore_wait(barrier_sem, N_DEV - 1)

            rdmas = []
            for off in range(1, N_DEV):
                p = lax.rem(my + off, N_DEV)
                rdma = pltpu.make_async_remote_copy(
                    src_ref=acc_ref,
                    dst_ref=comm_ref.at[my],
                    send_sem=send_sems.at[p],
                    recv_sem=recv_sems.at[my],
                    device_id=(p,),
                    device_id_type=pl.DeviceIdType.MESH,
                )
                rdma.start()
                rdmas.append(rdma)

            comm_ref[my, :, :] = acc_ref[:, :]

            for off in range(1, N_DEV):
                q = lax.rem(my + off, N_DEV)
                recv = pltpu.make_async_remote_copy(
                    src_ref=acc_ref,
                    dst_ref=comm_ref.at[q],
                    send_sem=send_sems.at[q],
                    recv_sem=recv_sems.at[q],
                    device_id=(q,),
                    device_id_type=pl.DeviceIdType.MESH,
                )
                recv.wait_recv()

            out_ref[:, :] = jnp.sum(comm_ref[:, :, :], axis=0)

            for rdma in rdmas:
                rdma.wait_send()

    return pl.pallas_call(
        body,
        grid=(G,),
        out_shape=jax.ShapeDtypeStruct((2, d), jnp.float32),
        in_specs=[
            pl.BlockSpec((mb, d), lambda g: (g, 0), memory_space=pltpu.VMEM),
            pl.BlockSpec((mb, d), lambda g: (g, 0), memory_space=pltpu.VMEM),
        ],
        out_specs=pl.BlockSpec((2, d), lambda g: (0, 0),
                               memory_space=pltpu.VMEM),
        scratch_shapes=[
            pltpu.VMEM((2, d), jnp.float32),
            pltpu.VMEM((N_DEV, 2, d), jnp.float32),
            pltpu.SemaphoreType.DMA((N_DEV,)),
            pltpu.SemaphoreType.DMA((N_DEV,)),
        ],
        compiler_params=pltpu.CompilerParams(collective_id=0),
    )(x, dy)
